# Initial kernel scaffold; baseline (speedup 1.0000x reference)
import jax
import jax.numpy as jnp
from jax.experimental import pallas as pl


def kernel(logits, val_freqs):
    raise NotImplementedError("write your pallas kernel here")


# speedup vs baseline: 467.5382x; 467.5382x over previous
import functools

import jax
import jax.numpy as jnp
from jax import lax
from jax.experimental import pallas as pl
from jax.experimental.pallas import tpu as pltpu
from jax.experimental.pallas import tpu_sc as plsc

L = 16


def _make_sc_kernel(B, C, HW, num_bins):
    try:
        info = plsc.get_sparse_core_info()
        NC, NS = info.num_cores, info.num_subcores
    except Exception:
        NC, NS = 2, 16
    NW = NC * NS
    assert (B * HW) % NW == 0
    PW = (B * HW) // NW
    WPB = NW // B
    assert WPB * B == NW and WPB * PW == HW
    P = 4096
    while PW % P != 0:
        P //= 2
    NCHUNK = PW // P
    VV = ((num_bins + L - 1) // L) * L

    mesh = plsc.VectorSubcoreMesh(
        core_axis_name="c", subcore_axis_name="s", num_cores=NC, num_subcores=NS
    )

    @functools.partial(
        pl.kernel,
        out_type=jax.ShapeDtypeStruct((B * C * HW,), jnp.float32),
        mesh=mesh,
        compiler_params=pltpu.CompilerParams(needs_layout_passes=False),
        scratch_types=[
            pltpu.VMEM((C * P,), jnp.float32),
            pltpu.VMEM((C * P,), jnp.float32),
            pltpu.VMEM((C * VV,), jnp.float32),
        ],
    )
    def k(x_hbm, vf_hbm, out_hbm, in_v, out_v, vf_v):
        wid = lax.axis_index("s") * NC + lax.axis_index("c")
        b = wid // WPB
        q = wid % WPB
        pltpu.sync_copy(vf_hbm, vf_v)

        def chunk_body(i, _):
            base = q * PW + i * P
            for c in range(C):
                pltpu.sync_copy(
                    x_hbm.at[pl.ds((b * C + c) * HW + base, P)],
                    in_v.at[pl.ds(c * P, P)],
                )

            def vec_body(j, _):
                off = j * L
                ls = [in_v[pl.ds(c * P + off, L)] for c in range(C)]
                m = ls[0]
                for c in range(1, C):
                    m = jnp.maximum(m, ls[c])
                es = [jnp.exp(l - m) for l in ls]
                s = es[0]
                for c in range(1, C):
                    s = s + es[c]
                rs = 1.0 / s
                cals = []
                for c in range(C):
                    p = es[c] * rs
                    bi = jnp.minimum((p * num_bins).astype(jnp.int32),
                                     num_bins - 1)
                    cals.append(plsc.load_gather(vf_v, [bi + c * VV]))
                s2 = cals[0]
                for c in range(1, C):
                    s2 = s2 + cals[c]
                s2 = jnp.where(s2 == 0.0, 1.0, s2)
                rs2 = 1.0 / s2
                for c in range(C):
                    out_v[pl.ds(c * P + off, L)] = cals[c] * rs2
                return 0

            lax.fori_loop(0, P // L, vec_body, 0)
            for c in range(C):
                pltpu.sync_copy(
                    out_v.at[pl.ds(c * P, P)],
                    out_hbm.at[pl.ds((b * C + c) * HW + base, P)],
                )
            return 0

        lax.fori_loop(0, NCHUNK, chunk_body, 0)

    return k


def kernel(logits, val_freqs):
    B, C, H, W = logits.shape
    num_classes, num_bins = val_freqs.shape
    assert num_classes == C
    HW = H * W
    VV = ((num_bins + L - 1) // L) * L
    vf_pad = jnp.pad(val_freqs, ((0, 0), (0, VV - num_bins))).reshape(-1)
    k = _make_sc_kernel(B, C, HW, num_bins)
    out = k(logits.reshape(-1), vf_pad)
    return out.reshape(B, C, H, W)


# speedup vs baseline: 682.0916x; 1.4589x over previous
import functools

import jax
import jax.numpy as jnp
from jax import lax
from jax.experimental import pallas as pl
from jax.experimental.pallas import tpu as pltpu
from jax.experimental.pallas import tpu_sc as plsc

L = 16
UNROLL = 8


def _make_sc_kernel(B, C, HW, num_bins):
    try:
        info = plsc.get_sparse_core_info()
        NC, NS = info.num_cores, info.num_subcores
    except Exception:
        NC, NS = 2, 16
    NW = NC * NS
    assert (B * HW) % NW == 0
    PW = (B * HW) // NW
    WPB = NW // B
    assert WPB * B == NW and WPB * PW == HW
    P = 4096
    while PW % (2 * P) != 0:
        P //= 2
    NT = PW // (2 * P)
    VV = ((num_bins + L - 1) // L) * L

    mesh = plsc.VectorSubcoreMesh(
        core_axis_name="c", subcore_axis_name="s", num_cores=NC, num_subcores=NS
    )

    @functools.partial(
        pl.kernel,
        out_type=jax.ShapeDtypeStruct((B * C * HW,), jnp.float32),
        mesh=mesh,
        compiler_params=pltpu.CompilerParams(needs_layout_passes=False),
        scratch_types=[
            pltpu.VMEM((2 * C * P,), jnp.float32),
            pltpu.VMEM((2 * C * P,), jnp.float32),
            pltpu.VMEM((C * VV,), jnp.float32),
            pltpu.SemaphoreType.DMA,
            pltpu.SemaphoreType.DMA,
            pltpu.SemaphoreType.DMA,
            pltpu.SemaphoreType.DMA,
        ],
    )
    def k(x_hbm, vf_hbm, out_hbm, in_v, out_v, vf_v,
          sem_in0, sem_in1, sem_out0, sem_out1):
        wid = lax.axis_index("s") * NC + lax.axis_index("c")
        b = wid // WPB
        q = wid % WPB
        pltpu.sync_copy(vf_hbm, vf_v)
        sems_in = (sem_in0, sem_in1)
        sems_out = (sem_out0, sem_out1)

        def issue_in(i, buf):
            for c in range(C):
                pltpu.async_copy(
                    x_hbm.at[pl.ds((b * C + c) * HW + q * PW + i * P, P)],
                    in_v.at[pl.ds((buf * C + c) * P, P)],
                    sems_in[buf],
                )

        def issue_out(i, buf):
            for c in range(C):
                pltpu.async_copy(
                    out_v.at[pl.ds((buf * C + c) * P, P)],
                    out_hbm.at[pl.ds((b * C + c) * HW + q * PW + i * P, P)],
                    sems_out[buf],
                )

        def drain_in(buf):
            for c in range(C):
                pltpu.make_async_copy(
                    x_hbm.at[pl.ds(0, P)],
                    in_v.at[pl.ds((buf * C + c) * P, P)],
                    sems_in[buf],
                ).wait()

        def drain_out(buf):
            for c in range(C):
                pltpu.make_async_copy(
                    out_v.at[pl.ds((buf * C + c) * P, P)],
                    out_hbm.at[pl.ds(0, P)],
                    sems_out[buf],
                ).wait()

        def compute_chunk(buf):
            base = buf * C * P

            @plsc.parallel_loop(0, P, L, unroll=UNROLL)
            def body(off):
                ls = [in_v[pl.ds(base + c * P + off, L)] for c in range(C)]
                m = ls[0]
                for c in range(1, C):
                    m = jnp.maximum(m, ls[c])
                es = [jnp.exp(l - m) for l in ls]
                s = es[0]
                for c in range(1, C):
                    s = s + es[c]
                rs = 1.0 / s
                cals = []
                for c in range(C):
                    p = es[c] * rs
                    bi = jnp.minimum((p * num_bins).astype(jnp.int32),
                                     num_bins - 1)
                    cals.append(plsc.load_gather(vf_v, [bi + c * VV]))
                s2 = cals[0]
                for c in range(1, C):
                    s2 = s2 + cals[c]
                s2 = jnp.where(s2 == 0.0, 1.0, s2)
                rs2 = 1.0 / s2
                for c in range(C):
                    out_v[pl.ds(base + c * P + off, L)] = cals[c] * rs2

        issue_in(0, 0)
        issue_in(1, 1)

        def pair_body(t, _):
            for buf in range(2):
                i = 2 * t + buf
                drain_in(buf)

                @pl.when(t > 0)
                def _():
                    drain_out(buf)

                compute_chunk(buf)
                issue_out(i, buf)

                @pl.when(t < NT - 1)
                def _():
                    issue_in(i + 2, buf)

            return 0

        lax.fori_loop(0, NT, pair_body, 0)
        drain_out(0)
        drain_out(1)

    return k


def kernel(logits, val_freqs):
    B, C, H, W = logits.shape
    num_classes, num_bins = val_freqs.shape
    assert num_classes == C
    HW = H * W
    VV = ((num_bins + L - 1) // L) * L
    vf_pad = jnp.pad(val_freqs, ((0, 0), (0, VV - num_bins))).reshape(-1)
    k = _make_sc_kernel(B, C, HW, num_bins)
    out = k(logits.reshape(-1), vf_pad)
    return out.reshape(B, C, H, W)
